# baseline (speedup 1.0000x reference)
import functools

import jax
import jax.numpy as jnp
from jax import lax
from jax.experimental import pallas as pl
from jax.experimental.pallas import tpu as pltpu
from jax.experimental.pallas import tpu_sc as plsc

_BLK = 128


def _sc_lookup(n_fields, batch, vocab, dim, n_workers):
    b_per_w = batch // n_workers
    n_blk = b_per_w // _BLK
    mesh = plsc.VectorSubcoreMesh(core_axis_name="c", subcore_axis_name="s")

    @functools.partial(
        pl.kernel,
        out_type=jax.ShapeDtypeStruct((n_fields, dim, batch), jnp.float32),
        mesh=mesh,
        scratch_types=[
            pltpu.VMEM((n_fields, _BLK), jnp.int32),
            pltpu.VMEM((4, _BLK, 2 * dim), jnp.float32),
            pltpu.VMEM((2, dim, _BLK + 1), jnp.float32),
            pltpu.SemaphoreType.DMA,
            pltpu.SemaphoreType.DMA,
            pltpu.SemaphoreType.DMA,
            pltpu.SemaphoreType.DMA,
            pltpu.SemaphoreType.DMA,
            pltpu.SemaphoreType.DMA,
        ],
        compiler_params=pltpu.CompilerParams(needs_layout_passes=False),
    )
    def k(
        ids_hbm, table_hbm, out_hbm, ids_v, fetch_v, otile_v,
        g0, g1, g2, g3, w0, w1,
    ):
        nc = lax.axis_size("c")
        wid = lax.axis_index("s") * nc + lax.axis_index("c")
        gsem = (g0, g1, g2, g3)
        wsem = (w0, w1)
        iota16 = lax.iota(jnp.int32, 16)
        rows_g = [gi * 16 + iota16 for gi in range(dim // 16)]
        _JU = 4

        def transpose_tile(pb, wb):
            src = fetch_v.at[pb]
            dst = otile_v.at[wb]

            def j_body(j0, carry):
                jb = j0 * _JU
                for k in range(_JU):
                    cols = jnp.full((16,), 0, jnp.int32) + (jb + k)
                    for gi in range(dim // 16):
                        vals = src[jb + k, pl.ds(16 * gi, 16)]
                        plsc.store_scatter(dst, [rows_g[gi], cols], vals)
                return carry

            lax.fori_loop(0, _BLK // _JU, j_body, 0)

        n_buf = 4

        def fire(f, b0):
            return pltpu.async_copy(
                table_hbm.at[ids_v.at[f]], fetch_v.at[f % n_buf], gsem[f % n_buf]
            )

        def blk_body(blk, carry):
            b0 = wid * b_per_w + blk * _BLK
            pltpu.sync_copy(ids_hbm.at[:, pl.ds(b0, _BLK)], ids_v)
            gathers = [None] * n_buf
            writes = [None, None]
            for f in range(n_buf - 1):
                gathers[f % n_buf] = fire(f, b0)
            for f in range(n_fields):
                pb = f % n_buf
                if f + n_buf - 1 < n_fields:
                    gathers[(f + n_buf - 1) % n_buf] = fire(f + n_buf - 1, b0)
                gathers[pb].wait()
                wb = f % 2
                if writes[wb] is not None:
                    writes[wb].wait()
                transpose_tile(pb, wb)
                writes[wb] = pltpu.async_copy(
                    otile_v.at[wb, :, pl.ds(0, _BLK)],
                    out_hbm.at[f, :, pl.ds(b0, _BLK)],
                    wsem[wb],
                )
            for d in writes:
                if d is not None:
                    d.wait()
            return carry

        lax.fori_loop(0, n_blk, blk_body, 0)

    return k


def kernel(ids, table):
    batch, n_fields = ids.shape
    vocab, dim = table.shape
    table_pad = jnp.pad(table, ((0, 0), (0, dim)))
    ids_t = ids.T
    out_t = _sc_lookup(n_fields, batch, vocab, dim, 32)(ids_t, table_pad)
    return out_t.transpose(2, 0, 1)


# speedup vs baseline: 1.1547x; 1.1547x over previous
import functools

import jax
import jax.numpy as jnp
from jax import lax
from jax.experimental import pallas as pl
from jax.experimental.pallas import tpu as pltpu
from jax.experimental.pallas import tpu_sc as plsc

_BLK = 128


def _sc_lookup(n_fields, batch, vocab, dim, n_workers):
    b_per_w = batch // n_workers
    n_blk = b_per_w // _BLK
    mesh = plsc.VectorSubcoreMesh(core_axis_name="c", subcore_axis_name="s")

    @functools.partial(
        pl.kernel,
        out_type=jax.ShapeDtypeStruct((n_fields, dim, batch), jnp.float32),
        mesh=mesh,
        scratch_types=[
            pltpu.VMEM((n_fields, _BLK), jnp.int32),
            pltpu.VMEM((4, _BLK, 2 * dim), jnp.float32),
            pltpu.VMEM((2, dim, _BLK + 1), jnp.float32),
            pltpu.SemaphoreType.DMA,
            pltpu.SemaphoreType.DMA,
            pltpu.SemaphoreType.DMA,
            pltpu.SemaphoreType.DMA,
            pltpu.SemaphoreType.DMA,
            pltpu.SemaphoreType.DMA,
        ],
        compiler_params=pltpu.CompilerParams(needs_layout_passes=False),
    )
    def k(
        ids_hbm, table_hbm, out_hbm, ids_v, fetch_v, otile_v,
        g0, g1, g2, g3, w0, w1,
    ):
        nc = lax.axis_size("c")
        wid = lax.axis_index("s") * nc + lax.axis_index("c")
        gsem = (g0, g1, g2, g3)
        wsem = (w0, w1)
        iota16 = lax.iota(jnp.int32, 16)
        rows_g = [gi * 16 + iota16 for gi in range(dim // 16)]
        _JU = 4

        def transpose_tile(pb, wb):
            src = fetch_v.at[pb]
            dst = otile_v.at[wb]

            @plsc.parallel_loop(0, _BLK, step=_JU, unroll=2)
            def j_body(jb):
                for k in range(_JU):
                    cols = jnp.full((16,), 0, jnp.int32) + (jb + k)
                    for gi in range(dim // 16):
                        vals = src[jb + k, pl.ds(16 * gi, 16)]
                        plsc.store_scatter(dst, [rows_g[gi], cols], vals)

        n_buf = 4

        def fire(f, b0):
            return pltpu.async_copy(
                table_hbm.at[ids_v.at[f]], fetch_v.at[f % n_buf], gsem[f % n_buf]
            )

        def blk_body(blk, carry):
            b0 = wid * b_per_w + blk * _BLK
            pltpu.sync_copy(ids_hbm.at[:, pl.ds(b0, _BLK)], ids_v)
            gathers = [None] * n_buf
            writes = [None, None]
            for f in range(n_buf - 1):
                gathers[f % n_buf] = fire(f, b0)
            for f in range(n_fields):
                pb = f % n_buf
                if f + n_buf - 1 < n_fields:
                    gathers[(f + n_buf - 1) % n_buf] = fire(f + n_buf - 1, b0)
                gathers[pb].wait()
                wb = f % 2
                if writes[wb] is not None:
                    writes[wb].wait()
                transpose_tile(pb, wb)
                writes[wb] = pltpu.async_copy(
                    otile_v.at[wb, :, pl.ds(0, _BLK)],
                    out_hbm.at[f, :, pl.ds(b0, _BLK)],
                    wsem[wb],
                )
            for d in writes:
                if d is not None:
                    d.wait()
            return carry

        lax.fori_loop(0, n_blk, blk_body, 0)

    return k


def kernel(ids, table):
    batch, n_fields = ids.shape
    vocab, dim = table.shape
    table_pad = jnp.pad(table, ((0, 0), (0, dim)))
    ids_t = ids.T
    out_t = _sc_lookup(n_fields, batch, vocab, dim, 32)(ids_t, table_pad)
    return out_t.transpose(2, 0, 1)


# speedup vs baseline: 1.4517x; 1.2572x over previous
import functools

import jax
import jax.numpy as jnp
from jax import lax
from jax.experimental import pallas as pl
from jax.experimental.pallas import tpu as pltpu
from jax.experimental.pallas import tpu_sc as plsc

_BLK = 128


def _sc_lookup(n_fields, batch, vocab, dim, n_workers):
    b_per_w = batch // n_workers
    n_blk = b_per_w // _BLK
    mesh = plsc.VectorSubcoreMesh(core_axis_name="c", subcore_axis_name="s")

    @functools.partial(
        pl.kernel,
        out_type=jax.ShapeDtypeStruct((n_fields, dim, batch), jnp.float32),
        mesh=mesh,
        scratch_types=[
            pltpu.VMEM((n_fields, _BLK), jnp.int32),
            pltpu.VMEM((2, _BLK, 2 * dim), jnp.float32),
            pltpu.VMEM((2, dim, _BLK + 1), jnp.float32),
            pltpu.SemaphoreType.DMA,
            pltpu.SemaphoreType.DMA,
            pltpu.SemaphoreType.DMA,
            pltpu.SemaphoreType.DMA,
        ],
        compiler_params=pltpu.CompilerParams(needs_layout_passes=False),
    )
    def k(
        ids_hbm, table_hbm, out_hbm, ids_v, fetch_v, otile_v, g0, g1, w0, w1,
    ):
        nc = lax.axis_size("c")
        wid = lax.axis_index("s") * nc + lax.axis_index("c")
        gsem = (g0, g1)
        wsem = (w0, w1)
        iota16 = lax.iota(jnp.int32, 16)
        diag = [(d + iota16) % 16 for d in range(16)]

        def transpose_tile(pb, wb):
            src = fetch_v.at[pb]
            dst = otile_v.at[wb]

            @plsc.parallel_loop(0, _BLK, step=16, unroll=2)
            def j_body(jb):
                rows = jb + iota16
                for cb in range(0, dim, 16):
                    for d in range(16):
                        cols = cb + diag[d]
                        vals = plsc.load_gather(src, [rows, cols])
                        plsc.store_scatter(dst, [cols, rows], vals)

        def fire(f, par, b0):
            return pltpu.async_copy(
                table_hbm.at[ids_v.at[f]], fetch_v.at[par], gsem[par]
            )

        def drain_gather(par):
            pltpu.make_async_copy(
                table_hbm.at[pl.ds(0, _BLK)], fetch_v.at[par], gsem[par]
            ).wait()

        def drain_write(par, b0):
            pltpu.make_async_copy(
                otile_v.at[par, :, pl.ds(0, _BLK)],
                out_hbm.at[0, :, pl.ds(b0, _BLK)],
                wsem[par],
            ).wait()

        def blk_body(blk, carry):
            b0 = wid * b_per_w + blk * _BLK
            pltpu.sync_copy(ids_hbm.at[:, pl.ds(b0, _BLK)], ids_v)
            fire(0, 0, b0)
            fire(1, 1, b0)

            def f_body(t, carry2):
                for par in range(2):
                    f = 2 * t + par

                    @pl.when(t < n_fields // 2 - 1)
                    def _(f=f, par=par):
                        fire(f + 2, par, b0)

                    drain_gather(par)

                    @pl.when(t > 0)
                    def _(par=par):
                        drain_write(par, b0)

                    transpose_tile(par, par)
                    pltpu.async_copy(
                        otile_v.at[par, :, pl.ds(0, _BLK)],
                        out_hbm.at[f, :, pl.ds(b0, _BLK)],
                        wsem[par],
                    )
                return carry2

            lax.fori_loop(0, n_fields // 2, f_body, 0)
            drain_write(0, b0)
            drain_write(1, b0)
            return carry

        lax.fori_loop(0, n_blk, blk_body, 0)

    return k


def kernel(ids, table):
    batch, n_fields = ids.shape
    vocab, dim = table.shape
    table_pad = jnp.pad(table, ((0, 0), (0, dim)))
    ids_t = ids.T
    out_t = _sc_lookup(n_fields, batch, vocab, dim, 32)(ids_t, table_pad)
    return out_t.transpose(2, 0, 1)
